# Initial kernel scaffold; baseline (speedup 1.0000x reference)
import jax
import jax.numpy as jnp
from jax.experimental import pallas as pl


def kernel(x, edge_index, hyperedge_index, W1, b1, Wh1, bh1, W2, b2, Wh2, bh2, lp_W, lp_b):
    raise NotImplementedError("write your pallas kernel here")


# speedup vs baseline: 33.7457x; 33.7457x over previous
import functools

import jax
import jax.numpy as jnp
from jax import lax
from jax.experimental import pallas as pl
from jax.experimental.pallas import tpu as pltpu
from jax.experimental.pallas import tpu_sc as plsc

N = 10000
F = 128
FH = 64
NP = 10240
NC = 2
NS = 16
CH = 160
EP = NS * CH * 128
RPT = NP // NS
BLK = 1024


def _deg_body(dst_hbm, degp, dst_v, ones_v, zrow_v, deg_sh):
    cid = lax.axis_index("c")
    sid = lax.axis_index("s")
    for k in range(8):
        ones_v[pl.ds(k * 16, 16)] = jnp.full((16,), 1.0, jnp.float32)
    for k in range(RPT // 16):
        zrow_v[pl.ds(k * 16, 16)] = jnp.zeros((16,), jnp.float32)

    @pl.when(cid == 0)
    def _():
        pltpu.sync_copy(dst_hbm.at[sid], dst_v)
        pltpu.sync_copy(zrow_v, deg_sh.at[pl.ds(sid * RPT, RPT)])
        plsc.subcore_barrier()

        @pl.loop(0, CH, step=4)
        def _(jo):
            for b in range(4):
                pltpu.sync_copy(ones_v, deg_sh.at[dst_v.at[jo + b]], add=True)

        plsc.subcore_barrier()
        pltpu.sync_copy(deg_sh.at[pl.ds(sid * RPT, RPT)],
                        degp.at[0, pl.ds(sid * RPT, RPT)])


def _mp_body(g_hbm, src_hbm, dst_hbm, acc, src_v, dst_v, rows0, rows1,
             sem0, sem1, acc_sh):
    cid = lax.axis_index("c")
    sid = lax.axis_index("s")
    pltpu.sync_copy(src_hbm.at[cid, sid], src_v)
    pltpu.sync_copy(dst_hbm.at[sid], dst_v)
    pltpu.sync_copy(g_hbm.at[pl.ds(cid * NP + sid * RPT, RPT)],
                    acc_sh.at[pl.ds(sid * RPT, RPT)])
    plsc.subcore_barrier()

    pltpu.async_copy(g_hbm.at[src_v.at[0]], rows0, sem0)
    pltpu.async_copy(g_hbm.at[src_v.at[1]], rows1, sem1)

    @pl.loop(0, CH, step=2)
    def _(jo):
        for b, (rows, sem) in enumerate(((rows0, sem0), (rows1, sem1))):
            j = jo + b
            pltpu.make_async_copy(g_hbm.at[src_v.at[j]], rows, sem).wait()
            pltpu.sync_copy(rows, acc_sh.at[dst_v.at[j]], add=True)

            @pl.when(j + 2 < CH)
            def _():
                pltpu.async_copy(g_hbm.at[src_v.at[j + 2]], rows, sem)

    plsc.subcore_barrier()
    pltpu.sync_copy(acc_sh.at[pl.ds(sid * RPT, RPT)],
                    acc.at[cid, pl.ds(sid * RPT, RPT)])


@functools.cache
def _sc_kernels():
    mesh = plsc.VectorSubcoreMesh(core_axis_name="c", subcore_axis_name="s")
    sc_degree = pl.kernel(
        _deg_body,
        out_type=jax.ShapeDtypeStruct((1, NP), jnp.float32),
        mesh=mesh,
        scratch_types=[
            pltpu.VMEM((CH, 128), jnp.int32),
            pltpu.VMEM((128,), jnp.float32),
            pltpu.VMEM((RPT,), jnp.float32),
            pltpu.VMEM_SHARED((NP,), jnp.float32),
        ],
    )
    sc_message_pass = pl.kernel(
        _mp_body,
        out_type=jax.ShapeDtypeStruct((NC, NP, FH), jnp.float32),
        mesh=mesh,
        compiler_params=pltpu.CompilerParams(use_tc_tiling_on_sc=False),
        scratch_types=[
            pltpu.VMEM((CH, 128), jnp.int32),
            pltpu.VMEM((CH, 128), jnp.int32),
            pltpu.VMEM((128, FH), jnp.float32),
            pltpu.VMEM((128, FH), jnp.float32),
            pltpu.SemaphoreType.DMA,
            pltpu.SemaphoreType.DMA,
            pltpu.VMEM_SHARED((NP, FH), jnp.float32),
        ],
    )
    return sc_degree, sc_message_pass


def _split_store(g_ref, h):
    g_ref[0] = h[:, :FH]
    g_ref[1] = h[:, FH:]


def _tc1_body(x_ref, w_ref, degp_ref, g_ref):
    dinv = lax.rsqrt(1.0 + degp_ref[0, :])
    h = jnp.dot(x_ref[...], w_ref[...], preferred_element_type=jnp.float32)
    _split_store(g_ref, h * dinv[:, None])


def _tc2_body(acc_ref, degp_ref, wbd_ref, b1_ref, g2_ref):
    dinv = lax.rsqrt(1.0 + degp_ref[0, :])
    a = jnp.concatenate([acc_ref[0], acc_ref[1]], axis=1)
    x1 = jnp.maximum(a * dinv[:, None] + b1_ref[...], 0.0)
    h2 = jnp.dot(x1, wbd_ref[...], preferred_element_type=jnp.float32)
    _split_store(g2_ref, h2 * dinv[:, None])


def _tc3_body(acc_ref, degp_ref, b2_ref, lpw_ref, lpb_ref, out_ref):
    dinv = lax.rsqrt(1.0 + degp_ref[0, :])
    a = jnp.concatenate([acc_ref[0], acc_ref[1]], axis=1)
    x2 = a * dinv[:, None] + b2_ref[...]
    logits = jnp.dot(x2, lpw_ref[...],
                     preferred_element_type=jnp.float32) + lpb_ref[...]
    m = jnp.max(logits, axis=1, keepdims=True)
    lse = jnp.log(jnp.sum(jnp.exp(logits - m), axis=1, keepdims=True)) + m
    out_ref[...] = logits - lse


_ROW = pl.BlockSpec((BLK, F), lambda i: (i, 0))
_GSPLIT = pl.BlockSpec((NC, BLK, FH), lambda i: (0, i, 0))
_DEGP = pl.BlockSpec((1, BLK), lambda i: (0, i))
_WFULL = pl.BlockSpec((F, F), lambda i: (0, 0))
_BROW = pl.BlockSpec((1, F), lambda i: (0, 0))

_tc1 = pl.pallas_call(
    _tc1_body,
    grid=(NP // BLK,),
    in_specs=[_ROW, _WFULL, _DEGP],
    out_specs=_GSPLIT,
    out_shape=jax.ShapeDtypeStruct((NC, NP, FH), jnp.float32),
)

_tc2 = pl.pallas_call(
    _tc2_body,
    grid=(NP // BLK,),
    in_specs=[_GSPLIT, _DEGP, _WFULL, _BROW],
    out_specs=_GSPLIT,
    out_shape=jax.ShapeDtypeStruct((NC, NP, FH), jnp.float32),
)

_tc3 = pl.pallas_call(
    _tc3_body,
    grid=(NP // BLK,),
    in_specs=[_GSPLIT, _DEGP, _BROW,
              pl.BlockSpec((F, 64), lambda i: (0, 0)),
              pl.BlockSpec((1, 64), lambda i: (0, 0))],
    out_specs=pl.BlockSpec((BLK, 64), lambda i: (i, 0)),
    out_shape=jax.ShapeDtypeStruct((NP, 64), jnp.float32),
)


def kernel(x, edge_index, hyperedge_index, W1, b1, Wh1, bh1, W2, b2, Wh2,
           bh2, lp_W, lp_b):
    del hyperedge_index

    wcat1 = jnp.concatenate([W1, Wh1], axis=1)
    wbd2 = jnp.zeros((F, F), jnp.float32)
    wbd2 = wbd2.at[:64, :64].set(W2).at[64:, 64:].set(Wh2)
    bcat1 = jnp.concatenate([b1, bh1])[None, :]
    bcat2 = jnp.concatenate([b2, bh2])[None, :]

    x_pad = jnp.zeros((NP, F), jnp.float32).at[:N].set(x)

    npad = EP - edge_index.shape[1]
    pad_idx = N + (jnp.arange(npad, dtype=jnp.int32) % (NP - N))
    src_flat = jnp.concatenate([edge_index[0], pad_idx])
    src = jnp.stack([src_flat, src_flat + NP]).reshape(NC, NS, CH, 128)
    dst = jnp.concatenate([edge_index[1], pad_idx]).reshape(NS, CH, 128)

    sc_degree, sc_message_pass = _sc_kernels()
    degp = sc_degree(dst)

    g1 = _tc1(x_pad, wcat1, degp)
    acc1 = sc_message_pass(g1.reshape(NC * NP, FH), src, dst)
    g2 = _tc2(acc1, degp, wbd2, bcat1)
    acc2 = sc_message_pass(g2.reshape(NC * NP, FH), src, dst)
    out = _tc3(acc2, degp, bcat2, lp_W, lp_b[None, :])
    return out[:N]


# speedup vs baseline: 41.5645x; 1.2317x over previous
import functools

import jax
import jax.numpy as jnp
from jax import lax
from jax.experimental import pallas as pl
from jax.experimental.pallas import tpu as pltpu
from jax.experimental.pallas import tpu_sc as plsc

N = 10000
F = 128
FH = 64
NP = 10240
NC = 2
NS = 16
CH = 160
EP = NS * CH * 128
RPT = NP // NS
BLK = 1024


def _deg_body(dst_hbm, degp, dst_v, ones_v, zrow_v, deg_sh):
    cid = lax.axis_index("c")
    sid = lax.axis_index("s")
    for k in range(8):
        ones_v[pl.ds(k * 16, 16)] = jnp.full((16,), 1.0, jnp.float32)
    for k in range(RPT // 16):
        zrow_v[pl.ds(k * 16, 16)] = jnp.zeros((16,), jnp.float32)
    pltpu.sync_copy(dst_hbm.at[sid, pl.ds(cid * (CH // 2), CH // 2)], dst_v)
    pltpu.sync_copy(zrow_v, deg_sh.at[pl.ds(sid * RPT, RPT)])
    plsc.subcore_barrier()

    @pl.loop(0, CH // 2, step=4)
    def _(jo):
        for b in range(4):
            pltpu.sync_copy(ones_v, deg_sh.at[dst_v.at[jo + b]], add=True)

    plsc.subcore_barrier()
    pltpu.sync_copy(deg_sh.at[pl.ds(sid * RPT, RPT)],
                    degp.at[cid, pl.ds(sid * RPT, RPT)])


def _mp_body(g_hbm, src_hbm, dst_hbm, acc, src_v, dst_v, rows0, rows1,
             rows2, rows3, sem0, sem1, sem2, sem3, acc_sh):
    cid = lax.axis_index("c")
    sid = lax.axis_index("s")
    pltpu.sync_copy(src_hbm.at[cid, sid], src_v)
    pltpu.sync_copy(dst_hbm.at[sid], dst_v)
    pltpu.sync_copy(g_hbm.at[pl.ds(cid * NP + sid * RPT, RPT)],
                    acc_sh.at[pl.ds(sid * RPT, RPT)])
    plsc.subcore_barrier()

    ring = ((rows0, sem0), (rows1, sem1), (rows2, sem2), (rows3, sem3))
    for b in range(4):
        pltpu.async_copy(g_hbm.at[src_v.at[b]], ring[b][0], ring[b][1])

    @pl.loop(0, CH, step=4)
    def _(jo):
        for b, (rows, sem) in enumerate(ring):
            j = jo + b
            pltpu.make_async_copy(g_hbm.at[src_v.at[j]], rows, sem).wait()
            pltpu.sync_copy(rows, acc_sh.at[dst_v.at[j]], add=True)

            @pl.when(j + 4 < CH)
            def _():
                pltpu.async_copy(g_hbm.at[src_v.at[j + 4]], rows, sem)

    plsc.subcore_barrier()
    pltpu.sync_copy(acc_sh.at[pl.ds(sid * RPT, RPT)],
                    acc.at[cid, pl.ds(sid * RPT, RPT)])


@functools.cache
def _sc_kernels():
    mesh = plsc.VectorSubcoreMesh(core_axis_name="c", subcore_axis_name="s")
    sc_degree = pl.kernel(
        _deg_body,
        out_type=jax.ShapeDtypeStruct((NC, NP), jnp.float32),
        mesh=mesh,
        scratch_types=[
            pltpu.VMEM((CH // 2, 128), jnp.int32),
            pltpu.VMEM((128,), jnp.float32),
            pltpu.VMEM((RPT,), jnp.float32),
            pltpu.VMEM_SHARED((NP,), jnp.float32),
        ],
    )
    sc_message_pass = pl.kernel(
        _mp_body,
        out_type=jax.ShapeDtypeStruct((NC, NP, FH), jnp.float32),
        mesh=mesh,
        compiler_params=pltpu.CompilerParams(use_tc_tiling_on_sc=False),
        scratch_types=[
            pltpu.VMEM((CH, 128), jnp.int32),
            pltpu.VMEM((CH, 128), jnp.int32),
            pltpu.VMEM((128, FH), jnp.float32),
            pltpu.VMEM((128, FH), jnp.float32),
            pltpu.VMEM((128, FH), jnp.float32),
            pltpu.VMEM((128, FH), jnp.float32),
            pltpu.SemaphoreType.DMA,
            pltpu.SemaphoreType.DMA,
            pltpu.SemaphoreType.DMA,
            pltpu.SemaphoreType.DMA,
            pltpu.VMEM_SHARED((NP, FH), jnp.float32),
        ],
    )
    return sc_degree, sc_message_pass


def _split_store(g_ref, h):
    g_ref[0] = h[:, :FH]
    g_ref[1] = h[:, FH:]


def _tc1_body(x_ref, w_ref, degp_ref, g_ref):
    dinv = lax.rsqrt(1.0 + degp_ref[0, :] + degp_ref[1, :])
    h = jnp.dot(x_ref[...], w_ref[...], preferred_element_type=jnp.float32)
    _split_store(g_ref, h * dinv[:, None])


def _tc2_body(acc_ref, degp_ref, wbd_ref, b1_ref, g2_ref):
    dinv = lax.rsqrt(1.0 + degp_ref[0, :] + degp_ref[1, :])
    a = jnp.concatenate([acc_ref[0], acc_ref[1]], axis=1)
    x1 = jnp.maximum(a * dinv[:, None] + b1_ref[...], 0.0)
    h2 = jnp.dot(x1, wbd_ref[...], preferred_element_type=jnp.float32)
    _split_store(g2_ref, h2 * dinv[:, None])


def _tc3_body(acc_ref, degp_ref, b2_ref, lpw_ref, lpb_ref, out_ref):
    dinv = lax.rsqrt(1.0 + degp_ref[0, :] + degp_ref[1, :])
    a = jnp.concatenate([acc_ref[0], acc_ref[1]], axis=1)
    x2 = a * dinv[:, None] + b2_ref[...]
    logits = jnp.dot(x2, lpw_ref[...],
                     preferred_element_type=jnp.float32) + lpb_ref[...]
    m = jnp.max(logits, axis=1, keepdims=True)
    lse = jnp.log(jnp.sum(jnp.exp(logits - m), axis=1, keepdims=True)) + m
    out_ref[...] = logits - lse


_ROW = pl.BlockSpec((BLK, F), lambda i: (i, 0))
_GSPLIT = pl.BlockSpec((NC, BLK, FH), lambda i: (0, i, 0))
_DEGP = pl.BlockSpec((NC, BLK), lambda i: (0, i))
_WFULL = pl.BlockSpec((F, F), lambda i: (0, 0))
_BROW = pl.BlockSpec((1, F), lambda i: (0, 0))

_tc1 = pl.pallas_call(
    _tc1_body,
    grid=(NP // BLK,),
    in_specs=[_ROW, _WFULL, _DEGP],
    out_specs=_GSPLIT,
    out_shape=jax.ShapeDtypeStruct((NC, NP, FH), jnp.float32),
)

_tc2 = pl.pallas_call(
    _tc2_body,
    grid=(NP // BLK,),
    in_specs=[_GSPLIT, _DEGP, _WFULL, _BROW],
    out_specs=_GSPLIT,
    out_shape=jax.ShapeDtypeStruct((NC, NP, FH), jnp.float32),
)

_tc3 = pl.pallas_call(
    _tc3_body,
    grid=(NP // BLK,),
    in_specs=[_GSPLIT, _DEGP, _BROW,
              pl.BlockSpec((F, 64), lambda i: (0, 0)),
              pl.BlockSpec((1, 64), lambda i: (0, 0))],
    out_specs=pl.BlockSpec((BLK, 64), lambda i: (i, 0)),
    out_shape=jax.ShapeDtypeStruct((NP, 64), jnp.float32),
)


def kernel(x, edge_index, hyperedge_index, W1, b1, Wh1, bh1, W2, b2, Wh2,
           bh2, lp_W, lp_b):
    del hyperedge_index

    wcat1 = jnp.concatenate([W1, Wh1], axis=1)
    wbd2 = jnp.zeros((F, F), jnp.float32)
    wbd2 = wbd2.at[:64, :64].set(W2).at[64:, 64:].set(Wh2)
    bcat1 = jnp.concatenate([b1, bh1])[None, :]
    bcat2 = jnp.concatenate([b2, bh2])[None, :]

    x_pad = jnp.zeros((NP, F), jnp.float32).at[:N].set(x)

    npad = EP - edge_index.shape[1]
    pad_idx = N + (jnp.arange(npad, dtype=jnp.int32) % (NP - N))
    src_flat = jnp.concatenate([edge_index[0], pad_idx])
    src = jnp.stack([src_flat, src_flat + NP]).reshape(NC, NS, CH, 128)
    dst = jnp.concatenate([edge_index[1], pad_idx]).reshape(NS, CH, 128)

    sc_degree, sc_message_pass = _sc_kernels()
    degp = sc_degree(dst)

    g1 = _tc1(x_pad, wcat1, degp)
    acc1 = sc_message_pass(g1.reshape(NC * NP, FH), src, dst)
    g2 = _tc2(acc1, degp, wbd2, bcat1)
    acc2 = sc_message_pass(g2.reshape(NC * NP, FH), src, dst)
    out = _tc3(acc2, degp, bcat2, lp_W, lp_b[None, :])
    return out[:N]


# speedup vs baseline: 41.7466x; 1.0044x over previous
import functools

import jax
import jax.numpy as jnp
from jax import lax
from jax.experimental import pallas as pl
from jax.experimental.pallas import tpu as pltpu
from jax.experimental.pallas import tpu_sc as plsc

N = 10000
F = 128
FH = 64
NP = 10240
NC = 2
NS = 16
CH = 160
EP = NS * CH * 128
RPT = NP // NS
BLK = 1024


def _deg_body(dst_hbm, degp, dst_v, ones_v, zrow_v, deg_sh):
    cid = lax.axis_index("c")
    sid = lax.axis_index("s")
    for k in range(8):
        ones_v[pl.ds(k * 16, 16)] = jnp.full((16,), 1.0, jnp.float32)
    for k in range(RPT // 16):
        zrow_v[pl.ds(k * 16, 16)] = jnp.zeros((16,), jnp.float32)
    pltpu.sync_copy(dst_hbm.at[sid, pl.ds(cid * (CH // 2), CH // 2)], dst_v)
    pltpu.sync_copy(zrow_v, deg_sh.at[pl.ds(sid * RPT, RPT)])
    plsc.subcore_barrier()

    @pl.loop(0, CH // 2, step=4)
    def _(jo):
        for b in range(4):
            pltpu.sync_copy(ones_v, deg_sh.at[dst_v.at[jo + b]], add=True)

    plsc.subcore_barrier()
    pltpu.sync_copy(deg_sh.at[pl.ds(sid * RPT, RPT)],
                    degp.at[cid, pl.ds(sid * RPT, RPT)])


def _mp_body(g_hbm, src_hbm, dst_hbm, acc, src_v, dst_v, rows0, rows1,
             rows2, rows3, sem0, sem1, sem2, sem3, acc_sh):
    cid = lax.axis_index("c")
    sid = lax.axis_index("s")
    pltpu.sync_copy(src_hbm.at[cid, sid], src_v)
    pltpu.sync_copy(dst_hbm.at[sid], dst_v)
    pltpu.sync_copy(g_hbm.at[pl.ds(cid * NP + sid * RPT, RPT)],
                    acc_sh.at[pl.ds(sid * RPT, RPT)])
    plsc.subcore_barrier()

    ring = ((rows0, sem0), (rows1, sem1), (rows2, sem2), (rows3, sem3))
    for b in range(4):
        pltpu.async_copy(g_hbm.at[src_v.at[b]], ring[b][0], ring[b][1])

    @pl.loop(0, CH, step=4)
    def _(jo):
        for b, (rows, sem) in enumerate(ring):
            j = jo + b
            pltpu.make_async_copy(g_hbm.at[src_v.at[j]], rows, sem).wait()
            pltpu.sync_copy(rows, acc_sh.at[dst_v.at[j]], add=True)

            @pl.when(j + 4 < CH)
            def _():
                pltpu.async_copy(g_hbm.at[src_v.at[j + 4]], rows, sem)

    plsc.subcore_barrier()
    pltpu.sync_copy(acc_sh.at[pl.ds(sid * RPT, RPT)],
                    acc.at[cid, pl.ds(sid * RPT, RPT)])


@functools.cache
def _sc_kernels():
    mesh = plsc.VectorSubcoreMesh(core_axis_name="c", subcore_axis_name="s")
    sc_degree = pl.kernel(
        _deg_body,
        out_type=jax.ShapeDtypeStruct((NC, NP), jnp.float32),
        mesh=mesh,
        scratch_types=[
            pltpu.VMEM((CH // 2, 128), jnp.int32),
            pltpu.VMEM((128,), jnp.float32),
            pltpu.VMEM((RPT,), jnp.float32),
            pltpu.VMEM_SHARED((NP,), jnp.float32),
        ],
    )
    sc_message_pass = pl.kernel(
        _mp_body,
        out_type=jax.ShapeDtypeStruct((NC, NP, FH), jnp.float32),
        mesh=mesh,
        compiler_params=pltpu.CompilerParams(use_tc_tiling_on_sc=False),
        scratch_types=[
            pltpu.VMEM((CH, 128), jnp.int32),
            pltpu.VMEM((CH, 128), jnp.int32),
            pltpu.VMEM((128, FH), jnp.float32),
            pltpu.VMEM((128, FH), jnp.float32),
            pltpu.VMEM((128, FH), jnp.float32),
            pltpu.VMEM((128, FH), jnp.float32),
            pltpu.SemaphoreType.DMA,
            pltpu.SemaphoreType.DMA,
            pltpu.SemaphoreType.DMA,
            pltpu.SemaphoreType.DMA,
            pltpu.VMEM_SHARED((NP, FH), jnp.float32),
        ],
    )
    return sc_degree, sc_message_pass


def _split_store(g_ref, h):
    g_ref[0] = h[:, :FH]
    g_ref[1] = h[:, FH:]


def _tc1_body(x_ref, w_ref, degp_ref, g_ref):
    dinv = lax.rsqrt(1.0 + degp_ref[0, :] + degp_ref[1, :])
    h = jnp.dot(x_ref[...], w_ref[...], preferred_element_type=jnp.float32)
    _split_store(g_ref, h * dinv[:, None])


def _tc2_body(acc_ref, degp_ref, wbd_ref, b1_ref, g2_ref):
    dinv = lax.rsqrt(1.0 + degp_ref[0, :] + degp_ref[1, :])
    a = jnp.concatenate([acc_ref[0], acc_ref[1]], axis=1)
    x1 = jnp.maximum(a * dinv[:, None] + b1_ref[...], 0.0)
    h2 = jnp.dot(x1, wbd_ref[...], preferred_element_type=jnp.float32)
    _split_store(g2_ref, h2 * dinv[:, None])


def _tc3_body(acc_ref, degp_ref, b2_ref, lpw_ref, lpb_ref, out_ref):
    dinv = lax.rsqrt(1.0 + degp_ref[0, :] + degp_ref[1, :])
    a = jnp.concatenate([acc_ref[0], acc_ref[1]], axis=1)
    x2 = a * dinv[:, None] + b2_ref[...]
    logits = jnp.dot(x2, lpw_ref[...],
                     preferred_element_type=jnp.float32) + lpb_ref[...]
    m = jnp.max(logits, axis=1, keepdims=True)
    lse = jnp.log(jnp.sum(jnp.exp(logits - m), axis=1, keepdims=True)) + m
    out_ref[...] = logits - lse


_ROW = pl.BlockSpec((BLK, F), lambda i: (i, 0))
_GSPLIT = pl.BlockSpec((NC, BLK, FH), lambda i: (0, i, 0))
_DEGP = pl.BlockSpec((NC, BLK), lambda i: (0, i))
_WFULL = pl.BlockSpec((F, F), lambda i: (0, 0))
_BROW = pl.BlockSpec((1, F), lambda i: (0, 0))

_tc1 = pl.pallas_call(
    _tc1_body,
    grid=(NP // BLK,),
    in_specs=[pl.BlockSpec((BLK, F), lambda i: (i, 0)), _WFULL, _DEGP],
    out_specs=_GSPLIT,
    out_shape=jax.ShapeDtypeStruct((NC, NP, FH), jnp.float32),
)

_tc2 = pl.pallas_call(
    _tc2_body,
    grid=(NP // BLK,),
    in_specs=[_GSPLIT, _DEGP, _WFULL, _BROW],
    out_specs=_GSPLIT,
    out_shape=jax.ShapeDtypeStruct((NC, NP, FH), jnp.float32),
)

_tc3 = pl.pallas_call(
    _tc3_body,
    grid=(NP // BLK,),
    in_specs=[_GSPLIT, _DEGP, _BROW,
              pl.BlockSpec((F, 64), lambda i: (0, 0)),
              pl.BlockSpec((1, 64), lambda i: (0, 0))],
    out_specs=pl.BlockSpec((BLK, 64), lambda i: (i, 0)),
    out_shape=jax.ShapeDtypeStruct((NP, 64), jnp.float32),
)


def kernel(x, edge_index, hyperedge_index, W1, b1, Wh1, bh1, W2, b2, Wh2,
           bh2, lp_W, lp_b):
    del hyperedge_index

    wcat1 = jnp.concatenate([W1, Wh1], axis=1)
    wbd2 = jnp.zeros((F, F), jnp.float32)
    wbd2 = wbd2.at[:64, :64].set(W2).at[64:, 64:].set(Wh2)
    bcat1 = jnp.concatenate([b1, bh1])[None, :]
    bcat2 = jnp.concatenate([b2, bh2])[None, :]

    npad = EP - edge_index.shape[1]
    pad_idx = N + (jnp.arange(npad, dtype=jnp.int32) % (NP - N))
    src_flat = jnp.concatenate([edge_index[0], pad_idx])
    src = jnp.stack([src_flat, src_flat + NP]).reshape(NC, NS, CH, 128)
    dst = jnp.concatenate([edge_index[1], pad_idx]).reshape(NS, CH, 128)

    sc_degree, sc_message_pass = _sc_kernels()
    degp = sc_degree(dst)

    g1 = _tc1(x, wcat1, degp)
    acc1 = sc_message_pass(g1.reshape(NC * NP, FH), src, dst)
    g2 = _tc2(acc1, degp, wbd2, bcat1)
    acc2 = sc_message_pass(g2.reshape(NC * NP, FH), src, dst)
    out = _tc3(acc2, degp, bcat2, lp_W, lp_b[None, :])
    return out[:N]


# speedup vs baseline: 42.1457x; 1.0096x over previous
import functools

import jax
import jax.numpy as jnp
from jax import lax
from jax.experimental import pallas as pl
from jax.experimental.pallas import tpu as pltpu
from jax.experimental.pallas import tpu_sc as plsc

N = 10000
E = 320000
F = 128
FH = 64
NP = 10240
NC = 2
NS = 16
NCHUNK = E // 128
CPT = NCHUNK // NS
CREM = NCHUNK - CPT * NS
CPW = NCHUNK // (NS * NC)
WREM = NCHUNK - CPW * NS * NC
RPT = NP // NS
BLK = 1024


def _deg_body(dst_hbm, degp, dst_v, ones_v, zrow_v, deg_sh):
    cid = lax.axis_index("c")
    sid = lax.axis_index("s")
    wid = sid * NC + cid
    for k in range(8):
        ones_v[pl.ds(k * 16, 16)] = jnp.full((16,), 1.0, jnp.float32)
    for k in range(RPT // 16):
        zrow_v[pl.ds(k * 16, 16)] = jnp.zeros((16,), jnp.float32)
    pltpu.sync_copy(dst_hbm.at[pl.ds(wid * CPW, CPW)], dst_v.at[pl.ds(0, CPW)])

    @pl.when(wid < WREM)
    def _():
        pltpu.sync_copy(dst_hbm.at[NS * NC * CPW + wid], dst_v.at[CPW])

    pltpu.sync_copy(zrow_v, deg_sh.at[pl.ds(sid * RPT, RPT)])
    plsc.subcore_barrier()

    @pl.loop(0, CPW, step=2)
    def _(jo):
        for b in range(2):
            pltpu.sync_copy(ones_v, deg_sh.at[dst_v.at[jo + b]], add=True)

    @pl.when(wid < WREM)
    def _():
        pltpu.sync_copy(ones_v, deg_sh.at[dst_v.at[CPW]], add=True)

    plsc.subcore_barrier()
    pltpu.sync_copy(deg_sh.at[pl.ds(sid * RPT, RPT)],
                    degp.at[cid, pl.ds(sid * RPT, RPT)])


def _mp_pipeline(g_ref, src_v, dst_v, ring, acc_sh, sid):
    for b in range(4):
        pltpu.async_copy(g_ref.at[src_v.at[b]], ring[b][0], ring[b][1])

    @pl.loop(0, CPT, step=4)
    def _(jo):
        for b, (rows, sem) in enumerate(ring):
            j = jo + b
            pltpu.make_async_copy(g_ref.at[src_v.at[j]], rows, sem).wait()
            pltpu.sync_copy(rows, acc_sh.at[dst_v.at[j]], add=True)

            @pl.when(j + 4 < CPT)
            def _():
                pltpu.async_copy(g_ref.at[src_v.at[j + 4]], rows, sem)

    @pl.when(sid < CREM)
    def _():
        rows, sem = ring[0]
        pltpu.async_copy(g_ref.at[src_v.at[CPT]], rows, sem).wait()
        pltpu.sync_copy(rows, acc_sh.at[dst_v.at[CPT]], add=True)


def _mp_body(g0_hbm, g1_hbm, src_hbm, dst_hbm, acc, src_v, dst_v, rows0,
             rows1, rows2, rows3, sem0, sem1, sem2, sem3, acc_sh):
    cid = lax.axis_index("c")
    sid = lax.axis_index("s")
    pltpu.sync_copy(src_hbm.at[pl.ds(sid * CPT, CPT)], src_v.at[pl.ds(0, CPT)])
    pltpu.sync_copy(dst_hbm.at[pl.ds(sid * CPT, CPT)], dst_v.at[pl.ds(0, CPT)])

    @pl.when(sid < CREM)
    def _():
        pltpu.sync_copy(src_hbm.at[NS * CPT + sid], src_v.at[CPT])
        pltpu.sync_copy(dst_hbm.at[NS * CPT + sid], dst_v.at[CPT])

    ring = ((rows0, sem0), (rows1, sem1), (rows2, sem2), (rows3, sem3))
    @pl.when(cid == 0)
    def _():
        pltpu.sync_copy(g0_hbm.at[pl.ds(sid * RPT, RPT)],
                        acc_sh.at[pl.ds(sid * RPT, RPT)])
        plsc.subcore_barrier()
        _mp_pipeline(g0_hbm, src_v, dst_v, ring, acc_sh, sid)

    @pl.when(cid == 1)
    def _():
        pltpu.sync_copy(g1_hbm.at[pl.ds(sid * RPT, RPT)],
                        acc_sh.at[pl.ds(sid * RPT, RPT)])
        plsc.subcore_barrier()
        _mp_pipeline(g1_hbm, src_v, dst_v, ring, acc_sh, sid)

    plsc.subcore_barrier()
    pltpu.sync_copy(acc_sh.at[pl.ds(sid * RPT, RPT)],
                    acc.at[cid, pl.ds(sid * RPT, RPT)])


@functools.cache
def _sc_kernels():
    mesh = plsc.VectorSubcoreMesh(core_axis_name="c", subcore_axis_name="s")
    sc_degree = pl.kernel(
        _deg_body,
        out_type=jax.ShapeDtypeStruct((NC, NP), jnp.float32),
        mesh=mesh,
        compiler_params=pltpu.CompilerParams(use_tc_tiling_on_sc=False),
        scratch_types=[
            pltpu.VMEM((CPW + 1, 128), jnp.int32),
            pltpu.VMEM((128,), jnp.float32),
            pltpu.VMEM((RPT,), jnp.float32),
            pltpu.VMEM_SHARED((NP,), jnp.float32),
        ],
    )
    sc_message_pass = pl.kernel(
        _mp_body,
        out_type=jax.ShapeDtypeStruct((NC, NP, FH), jnp.float32),
        mesh=mesh,
        compiler_params=pltpu.CompilerParams(use_tc_tiling_on_sc=False),
        scratch_types=[
            pltpu.VMEM((CPT + 1, 128), jnp.int32),
            pltpu.VMEM((CPT + 1, 128), jnp.int32),
            pltpu.VMEM((128, FH), jnp.float32),
            pltpu.VMEM((128, FH), jnp.float32),
            pltpu.VMEM((128, FH), jnp.float32),
            pltpu.VMEM((128, FH), jnp.float32),
            pltpu.SemaphoreType.DMA,
            pltpu.SemaphoreType.DMA,
            pltpu.SemaphoreType.DMA,
            pltpu.SemaphoreType.DMA,
            pltpu.VMEM_SHARED((NP, FH), jnp.float32),
        ],
    )
    return sc_degree, sc_message_pass


def _split_store(g_ref0, g_ref1, h):
    g_ref0[...] = h[:, :FH]
    g_ref1[...] = h[:, FH:]


def _tc1_body(x_ref, w_ref, degp_ref, g0_ref, g1_ref):
    dinv = lax.rsqrt(1.0 + degp_ref[0, :] + degp_ref[1, :])
    h = jnp.dot(x_ref[...], w_ref[...], preferred_element_type=jnp.float32)
    _split_store(g0_ref, g1_ref, h * dinv[:, None])


def _tc2_body(acc_ref, degp_ref, wbd_ref, b1_ref, g0_ref, g1_ref):
    dinv = lax.rsqrt(1.0 + degp_ref[0, :] + degp_ref[1, :])
    a = jnp.concatenate([acc_ref[0], acc_ref[1]], axis=1)
    x1 = jnp.maximum(a * dinv[:, None] + b1_ref[...], 0.0)
    h2 = jnp.dot(x1, wbd_ref[...], preferred_element_type=jnp.float32)
    _split_store(g0_ref, g1_ref, h2 * dinv[:, None])


def _tc3_body(acc_ref, degp_ref, b2_ref, lpw_ref, lpb_ref, out_ref):
    dinv = lax.rsqrt(1.0 + degp_ref[0, :] + degp_ref[1, :])
    a = jnp.concatenate([acc_ref[0], acc_ref[1]], axis=1)
    x2 = a * dinv[:, None] + b2_ref[...]
    logits = jnp.dot(x2, lpw_ref[...],
                     preferred_element_type=jnp.float32) + lpb_ref[...]
    m = jnp.max(logits, axis=1, keepdims=True)
    lse = jnp.log(jnp.sum(jnp.exp(logits - m), axis=1, keepdims=True)) + m
    out_ref[...] = logits - lse


_GHALF = pl.BlockSpec((BLK, FH), lambda i: (i, 0))
_ACC = pl.BlockSpec((NC, BLK, FH), lambda i: (0, i, 0))
_DEGP = pl.BlockSpec((NC, BLK), lambda i: (0, i))
_WFULL = pl.BlockSpec((F, F), lambda i: (0, 0))
_BROW = pl.BlockSpec((1, F), lambda i: (0, 0))
_GOUT = jax.ShapeDtypeStruct((NP, FH), jnp.float32)

_tc1 = pl.pallas_call(
    _tc1_body,
    grid=(NP // BLK,),
    in_specs=[pl.BlockSpec((BLK, F), lambda i: (i, 0)), _WFULL, _DEGP],
    out_specs=[_GHALF, _GHALF],
    out_shape=[_GOUT, _GOUT],
)

_tc2 = pl.pallas_call(
    _tc2_body,
    grid=(NP // BLK,),
    in_specs=[_ACC, _DEGP, _WFULL, _BROW],
    out_specs=[_GHALF, _GHALF],
    out_shape=[_GOUT, _GOUT],
)

_tc3 = pl.pallas_call(
    _tc3_body,
    grid=(NP // BLK,),
    in_specs=[_ACC, _DEGP, _BROW,
              pl.BlockSpec((F, 64), lambda i: (0, 0)),
              pl.BlockSpec((1, 64), lambda i: (0, 0))],
    out_specs=pl.BlockSpec((BLK, 64), lambda i: (i, 0)),
    out_shape=jax.ShapeDtypeStruct((NP, 64), jnp.float32),
)


def kernel(x, edge_index, hyperedge_index, W1, b1, Wh1, bh1, W2, b2, Wh2,
           bh2, lp_W, lp_b):
    del hyperedge_index

    wcat1 = jnp.concatenate([W1, Wh1], axis=1)
    wbd2 = jnp.zeros((F, F), jnp.float32)
    wbd2 = wbd2.at[:64, :64].set(W2).at[64:, 64:].set(Wh2)
    bcat1 = jnp.concatenate([b1, bh1])[None, :]
    bcat2 = jnp.concatenate([b2, bh2])[None, :]

    src = edge_index[0].reshape(NCHUNK, 128)
    dst = edge_index[1].reshape(NCHUNK, 128)

    sc_degree, sc_message_pass = _sc_kernels()
    degp = sc_degree(dst)

    g1a, g1b = _tc1(x, wcat1, degp)
    acc1 = sc_message_pass(g1a, g1b, src, dst)
    g2a, g2b = _tc2(acc1, degp, wbd2, bcat1)
    acc2 = sc_message_pass(g2a, g2b, src, dst)
    out = _tc3(acc2, degp, bcat2, lp_W, lp_b[None, :])
    return out[:N]


# speedup vs baseline: 42.1825x; 1.0009x over previous
import functools

import jax
import jax.numpy as jnp
from jax import lax
from jax.experimental import pallas as pl
from jax.experimental.pallas import tpu as pltpu
from jax.experimental.pallas import tpu_sc as plsc

N = 10000
E = 320000
F = 128
FH = 64
NP = 10240
NC = 2
NS = 16
NCHUNK = E // 128
CPT = NCHUNK // NS
CREM = NCHUNK - CPT * NS
CPW = NCHUNK // (NS * NC)
WREM = NCHUNK - CPW * NS * NC
RPT = NP // NS
BLK = 1024


def _deg_body(dst_hbm, degp, dst_v, ones_v, zrow_v, deg_sh):
    cid = lax.axis_index("c")
    sid = lax.axis_index("s")
    wid = sid * NC + cid
    for k in range(8):
        ones_v[pl.ds(k * 16, 16)] = jnp.full((16,), 1.0, jnp.float32)
    for k in range(RPT // 16):
        zrow_v[pl.ds(k * 16, 16)] = jnp.zeros((16,), jnp.float32)
    pltpu.sync_copy(dst_hbm.at[pl.ds(wid * CPW, CPW)], dst_v.at[pl.ds(0, CPW)])

    @pl.when(wid < WREM)
    def _():
        pltpu.sync_copy(dst_hbm.at[NS * NC * CPW + wid], dst_v.at[CPW])

    pltpu.sync_copy(zrow_v, deg_sh.at[pl.ds(sid * RPT, RPT)])
    plsc.subcore_barrier()

    @pl.loop(0, CPW, step=2)
    def _(jo):
        for b in range(2):
            pltpu.sync_copy(ones_v, deg_sh.at[dst_v.at[jo + b]], add=True)

    @pl.when(wid < WREM)
    def _():
        pltpu.sync_copy(ones_v, deg_sh.at[dst_v.at[CPW]], add=True)

    plsc.subcore_barrier()
    pltpu.sync_copy(deg_sh.at[pl.ds(sid * RPT, RPT)],
                    degp.at[cid, pl.ds(sid * RPT, RPT)])


def _mp_pipeline(gsub, src_v, dst_v, ring, acc_sh, sid):
    pltpu.sync_copy(gsub.at[pl.ds(sid * RPT, RPT)],
                    acc_sh.at[pl.ds(sid * RPT, RPT)])
    plsc.subcore_barrier()

    for b in range(4):
        pltpu.async_copy(gsub.at[src_v.at[b]], ring[b][0], ring[b][1])

    @pl.loop(0, CPT, step=4)
    def _(jo):
        for b, (rows, sem) in enumerate(ring):
            j = jo + b
            pltpu.make_async_copy(gsub.at[src_v.at[j]], rows, sem).wait()
            pltpu.sync_copy(rows, acc_sh.at[dst_v.at[j]], add=True)

            @pl.when(j + 4 < CPT)
            def _():
                pltpu.async_copy(gsub.at[src_v.at[j + 4]], rows, sem)

    @pl.when(sid < CREM)
    def _():
        rows, sem = ring[0]
        pltpu.async_copy(gsub.at[src_v.at[CPT]], rows, sem).wait()
        pltpu.sync_copy(rows, acc_sh.at[dst_v.at[CPT]], add=True)


def _mp_body(g0_hbm, g1_hbm, src_hbm, dst_hbm, acc, src_v, dst_v, rows0,
             rows1, rows2, rows3, sem0, sem1, sem2, sem3, acc_sh):
    cid = lax.axis_index("c")
    sid = lax.axis_index("s")
    pltpu.sync_copy(src_hbm.at[pl.ds(sid * CPT, CPT)], src_v.at[pl.ds(0, CPT)])
    pltpu.sync_copy(dst_hbm.at[pl.ds(sid * CPT, CPT)], dst_v.at[pl.ds(0, CPT)])

    @pl.when(sid < CREM)
    def _():
        pltpu.sync_copy(src_hbm.at[NS * CPT + sid], src_v.at[CPT])
        pltpu.sync_copy(dst_hbm.at[NS * CPT + sid], dst_v.at[CPT])

    ring = ((rows0, sem0), (rows1, sem1), (rows2, sem2), (rows3, sem3))

    @pl.when(cid == 0)
    def _():
        _mp_pipeline(g0_hbm, src_v, dst_v, ring, acc_sh, sid)

    @pl.when(cid == 1)
    def _():
        _mp_pipeline(g1_hbm, src_v, dst_v, ring, acc_sh, sid)

    plsc.subcore_barrier()
    pltpu.sync_copy(acc_sh.at[pl.ds(sid * RPT, RPT)],
                    acc.at[cid, pl.ds(sid * RPT, RPT)])


@functools.cache
def _sc_kernels():
    mesh = plsc.VectorSubcoreMesh(core_axis_name="c", subcore_axis_name="s")
    sc_degree = pl.kernel(
        _deg_body,
        out_type=jax.ShapeDtypeStruct((NC, NP), jnp.float32),
        mesh=mesh,
        compiler_params=pltpu.CompilerParams(use_tc_tiling_on_sc=False),
        scratch_types=[
            pltpu.VMEM((CPW + 1, 128), jnp.int32),
            pltpu.VMEM((128,), jnp.float32),
            pltpu.VMEM((RPT,), jnp.float32),
            pltpu.VMEM_SHARED((NP,), jnp.float32),
        ],
    )
    sc_message_pass = pl.kernel(
        _mp_body,
        out_type=jax.ShapeDtypeStruct((NC, NP, FH), jnp.float32),
        mesh=mesh,
        compiler_params=pltpu.CompilerParams(use_tc_tiling_on_sc=False),
        scratch_types=[
            pltpu.VMEM((CPT + 1, 128), jnp.int32),
            pltpu.VMEM((CPT + 1, 128), jnp.int32),
            pltpu.VMEM((128, FH), jnp.float32),
            pltpu.VMEM((128, FH), jnp.float32),
            pltpu.VMEM((128, FH), jnp.float32),
            pltpu.VMEM((128, FH), jnp.float32),
            pltpu.SemaphoreType.DMA,
            pltpu.SemaphoreType.DMA,
            pltpu.SemaphoreType.DMA,
            pltpu.SemaphoreType.DMA,
            pltpu.VMEM_SHARED((NP, FH), jnp.float32),
        ],
    )
    return sc_degree, sc_message_pass


def _tc1a_body(x_ref, w_ref, h_ref):
    h_ref[...] = jnp.dot(x_ref[...], w_ref[...],
                         preferred_element_type=jnp.float32)


def _tc1b_body(h_ref, degp_ref, g0_ref, g1_ref):
    dinv = lax.rsqrt(1.0 + degp_ref[0, :] + degp_ref[1, :])
    g = h_ref[...] * dinv[:, None]
    g0_ref[...] = g[:, :FH]
    g1_ref[...] = g[:, FH:]


def _tc2_body(acc_ref, degp_ref, wbd_ref, b1_ref, g0_ref, g1_ref):
    dinv = lax.rsqrt(1.0 + degp_ref[0, :] + degp_ref[1, :])
    a = jnp.concatenate([acc_ref[0], acc_ref[1]], axis=1)
    x1 = jnp.maximum(a * dinv[:, None] + b1_ref[...], 0.0)
    h2 = jnp.dot(x1, wbd_ref[...], preferred_element_type=jnp.float32)
    g = h2 * dinv[:, None]
    g0_ref[...] = g[:, :FH]
    g1_ref[...] = g[:, FH:]


def _tc3_body(acc_ref, degp_ref, b2_ref, lpw_ref, lpb_ref, out_ref):
    dinv = lax.rsqrt(1.0 + degp_ref[0, :] + degp_ref[1, :])
    a = jnp.concatenate([acc_ref[0], acc_ref[1]], axis=1)
    x2 = a * dinv[:, None] + b2_ref[...]
    logits = jnp.dot(x2, lpw_ref[...],
                     preferred_element_type=jnp.float32) + lpb_ref[...]
    m = jnp.max(logits, axis=1, keepdims=True)
    lse = jnp.log(jnp.sum(jnp.exp(logits - m), axis=1, keepdims=True)) + m
    out_ref[...] = logits - lse


_ROW = pl.BlockSpec((BLK, F), lambda i: (i, 0))
_ACC = pl.BlockSpec((NC, BLK, FH), lambda i: (0, i, 0))
_DEGP = pl.BlockSpec((NC, BLK), lambda i: (0, i))
_WFULL = pl.BlockSpec((F, F), lambda i: (0, 0))
_BROW = pl.BlockSpec((1, F), lambda i: (0, 0))
_GHALF = pl.BlockSpec((BLK, FH), lambda i: (i, 0))
_HOUT = jax.ShapeDtypeStruct((NP, F), jnp.float32)
_GOUT = jax.ShapeDtypeStruct((NP, FH), jnp.float32)

_tc1a = pl.pallas_call(
    _tc1a_body,
    grid=(NP // BLK,),
    in_specs=[pl.BlockSpec((BLK, F), lambda i: (i, 0)), _WFULL],
    out_specs=_ROW,
    out_shape=_HOUT,
)

_tc1b = pl.pallas_call(
    _tc1b_body,
    grid=(NP // BLK,),
    in_specs=[_ROW, _DEGP],
    out_specs=[_GHALF, _GHALF],
    out_shape=[_GOUT, _GOUT],
)

_tc2 = pl.pallas_call(
    _tc2_body,
    grid=(NP // BLK,),
    in_specs=[_ACC, _DEGP, _WFULL, _BROW],
    out_specs=[_GHALF, _GHALF],
    out_shape=[_GOUT, _GOUT],
)

_tc3 = pl.pallas_call(
    _tc3_body,
    grid=(NP // BLK,),
    in_specs=[_ACC, _DEGP, _BROW,
              pl.BlockSpec((F, 64), lambda i: (0, 0)),
              pl.BlockSpec((1, 64), lambda i: (0, 0))],
    out_specs=pl.BlockSpec((BLK, 64), lambda i: (i, 0)),
    out_shape=jax.ShapeDtypeStruct((NP, 64), jnp.float32),
)


def kernel(x, edge_index, hyperedge_index, W1, b1, Wh1, bh1, W2, b2, Wh2,
           bh2, lp_W, lp_b):
    del hyperedge_index

    wcat1 = jnp.concatenate([W1, Wh1], axis=1)
    wbd2 = jnp.zeros((F, F), jnp.float32)
    wbd2 = wbd2.at[:64, :64].set(W2).at[64:, 64:].set(Wh2)
    bcat1 = jnp.concatenate([b1, bh1])[None, :]
    bcat2 = jnp.concatenate([b2, bh2])[None, :]

    src = edge_index[0].reshape(NCHUNK, 128)
    dst = edge_index[1].reshape(NCHUNK, 128)

    sc_degree, sc_message_pass = _sc_kernels()
    degp = sc_degree(dst)

    h1 = _tc1a(x, wcat1)
    g1a, g1b = _tc1b(h1, degp)
    acc1 = sc_message_pass(g1a, g1b, src, dst)
    g2a, g2b = _tc2(acc1, degp, wbd2, bcat1)
    acc2 = sc_message_pass(g2a, g2b, src, dst)
    out = _tc3(acc2, degp, bcat2, lp_W, lp_b[None, :])
    return out[:N]


# speedup vs baseline: 45.3750x; 1.0757x over previous
import functools

import jax
import jax.numpy as jnp
from jax import lax
from jax.experimental import pallas as pl
from jax.experimental.pallas import tpu as pltpu
from jax.experimental.pallas import tpu_sc as plsc

N = 10000
E = 320000
F = 128
FH = 64
NP = 10240
NC = 2
NS = 16
NCHUNK = E // 128
CPT = NCHUNK // NS
CREM = NCHUNK - CPT * NS
CPW = NCHUNK // (NS * NC)
WREM = NCHUNK - CPW * NS * NC
RPT = NP // NS
BLK = 1024


def _deg_body(dst_hbm, degp, dst_v, ones_v, zrow_v, deg_sh):
    cid = lax.axis_index("c")
    sid = lax.axis_index("s")
    wid = sid * NC + cid
    for k in range(8):
        ones_v[pl.ds(k * 16, 16)] = jnp.full((16,), 1.0, jnp.float32)
    for k in range(RPT // 16):
        zrow_v[pl.ds(k * 16, 16)] = jnp.zeros((16,), jnp.float32)
    pltpu.sync_copy(dst_hbm.at[pl.ds(wid * CPW, CPW)], dst_v.at[pl.ds(0, CPW)])

    @pl.when(wid < WREM)
    def _():
        pltpu.sync_copy(dst_hbm.at[NS * NC * CPW + wid], dst_v.at[CPW])

    pltpu.sync_copy(zrow_v, deg_sh.at[pl.ds(sid * RPT, RPT)])
    plsc.subcore_barrier()

    @pl.loop(0, CPW, step=2)
    def _(jo):
        for b in range(2):
            pltpu.sync_copy(ones_v, deg_sh.at[dst_v.at[jo + b]], add=True)

    @pl.when(wid < WREM)
    def _():
        pltpu.sync_copy(ones_v, deg_sh.at[dst_v.at[CPW]], add=True)

    plsc.subcore_barrier()
    pltpu.sync_copy(deg_sh.at[pl.ds(sid * RPT, RPT)],
                    degp.at[cid, pl.ds(sid * RPT, RPT)])


def _mp_pipeline(gsub, src_v, dst_v, ring, acc_sh, sid):
    pltpu.sync_copy(gsub.at[pl.ds(sid * RPT, RPT)],
                    acc_sh.at[pl.ds(sid * RPT, RPT)])
    plsc.subcore_barrier()

    for b in range(4):
        pltpu.async_copy(gsub.at[src_v.at[b]], ring[b][0], ring[b][1])

    @pl.loop(0, CPT, step=4)
    def _(jo):
        for b, (rows, sem) in enumerate(ring):
            j = jo + b
            pltpu.make_async_copy(gsub.at[src_v.at[j]], rows, sem).wait()
            pltpu.sync_copy(rows, acc_sh.at[dst_v.at[j]], add=True)

            @pl.when(j + 4 < CPT)
            def _():
                pltpu.async_copy(gsub.at[src_v.at[j + 4]], rows, sem)

    @pl.when(sid < CREM)
    def _():
        rows, sem = ring[0]
        pltpu.async_copy(gsub.at[src_v.at[CPT]], rows, sem).wait()
        pltpu.sync_copy(rows, acc_sh.at[dst_v.at[CPT]], add=True)


def _mp_body(g0_hbm, g1_hbm, src_hbm, dst_hbm, acc, src_v, dst_v, rows0,
             rows1, rows2, rows3, sem0, sem1, sem2, sem3, acc_sh):
    cid = lax.axis_index("c")
    sid = lax.axis_index("s")
    pltpu.sync_copy(src_hbm.at[pl.ds(sid * CPT, CPT)], src_v.at[pl.ds(0, CPT)])
    pltpu.sync_copy(dst_hbm.at[pl.ds(sid * CPT, CPT)], dst_v.at[pl.ds(0, CPT)])

    @pl.when(sid < CREM)
    def _():
        pltpu.sync_copy(src_hbm.at[NS * CPT + sid], src_v.at[CPT])
        pltpu.sync_copy(dst_hbm.at[NS * CPT + sid], dst_v.at[CPT])

    ring = ((rows0, sem0), (rows1, sem1), (rows2, sem2), (rows3, sem3))

    @pl.when(cid == 0)
    def _():
        _mp_pipeline(g0_hbm, src_v, dst_v, ring, acc_sh, sid)

    @pl.when(cid == 1)
    def _():
        _mp_pipeline(g1_hbm, src_v, dst_v, ring, acc_sh, sid)

    plsc.subcore_barrier()
    pltpu.sync_copy(acc_sh.at[pl.ds(sid * RPT, RPT)],
                    acc.at[pl.ds(sid * RPT, RPT), pl.ds(cid * FH, FH)])


@functools.cache
def _sc_kernels():
    mesh = plsc.VectorSubcoreMesh(core_axis_name="c", subcore_axis_name="s")
    sc_degree = pl.kernel(
        _deg_body,
        out_type=jax.ShapeDtypeStruct((NC, NP), jnp.float32),
        mesh=mesh,
        compiler_params=pltpu.CompilerParams(use_tc_tiling_on_sc=False),
        scratch_types=[
            pltpu.VMEM((CPW + 1, 128), jnp.int32),
            pltpu.VMEM((128,), jnp.float32),
            pltpu.VMEM((RPT,), jnp.float32),
            pltpu.VMEM_SHARED((NP,), jnp.float32),
        ],
    )
    sc_message_pass = pl.kernel(
        _mp_body,
        out_type=jax.ShapeDtypeStruct((NP, F), jnp.float32),
        mesh=mesh,
        compiler_params=pltpu.CompilerParams(use_tc_tiling_on_sc=False),
        scratch_types=[
            pltpu.VMEM((CPT + 1, 128), jnp.int32),
            pltpu.VMEM((CPT + 1, 128), jnp.int32),
            pltpu.VMEM((128, FH), jnp.float32),
            pltpu.VMEM((128, FH), jnp.float32),
            pltpu.VMEM((128, FH), jnp.float32),
            pltpu.VMEM((128, FH), jnp.float32),
            pltpu.SemaphoreType.DMA,
            pltpu.SemaphoreType.DMA,
            pltpu.SemaphoreType.DMA,
            pltpu.SemaphoreType.DMA,
            pltpu.VMEM_SHARED((NP, FH), jnp.float32),
        ],
    )
    return sc_degree, sc_message_pass


def _tc1a_body(x_ref, w_ref, h_ref):
    h_ref[...] = jnp.dot(x_ref[...], w_ref[...],
                         preferred_element_type=jnp.float32)


def _tc1b_body(h_ref, degp_ref, g0_ref, g1_ref):
    dinv = lax.rsqrt(1.0 + degp_ref[0, :] + degp_ref[1, :])
    g = h_ref[...] * dinv[:, None]
    g0_ref[...] = g[:, :FH]
    g1_ref[...] = g[:, FH:]


def _tc2_body(acc_ref, degp_ref, wbd_ref, b1_ref, g0_ref, g1_ref):
    dinv = lax.rsqrt(1.0 + degp_ref[0, :] + degp_ref[1, :])
    x1 = jnp.maximum(acc_ref[...] * dinv[:, None] + b1_ref[...], 0.0)
    h2 = jnp.dot(x1, wbd_ref[...], preferred_element_type=jnp.float32)
    g = h2 * dinv[:, None]
    g0_ref[...] = g[:, :FH]
    g1_ref[...] = g[:, FH:]


def _tc3_body(acc_ref, degp_ref, b2_ref, lpw_ref, lpb_ref, out_ref):
    dinv = lax.rsqrt(1.0 + degp_ref[0, :] + degp_ref[1, :])
    x2 = acc_ref[...] * dinv[:, None] + b2_ref[...]
    logits = jnp.dot(x2, lpw_ref[...],
                     preferred_element_type=jnp.float32) + lpb_ref[...]
    m = jnp.max(logits, axis=1, keepdims=True)
    lse = jnp.log(jnp.sum(jnp.exp(logits - m), axis=1, keepdims=True)) + m
    out_ref[...] = logits - lse


_ROW = pl.BlockSpec((BLK, F), lambda i: (i, 0))
_ACC = pl.BlockSpec((NC, BLK, FH), lambda i: (0, i, 0))
_DEGP = pl.BlockSpec((NC, BLK), lambda i: (0, i))
_WFULL = pl.BlockSpec((F, F), lambda i: (0, 0))
_BROW = pl.BlockSpec((1, F), lambda i: (0, 0))
_GHALF = pl.BlockSpec((BLK, FH), lambda i: (i, 0))
_HOUT = jax.ShapeDtypeStruct((NP, F), jnp.float32)
_GOUT = jax.ShapeDtypeStruct((NP, FH), jnp.float32)

_tc1a = pl.pallas_call(
    _tc1a_body,
    grid=(NP // BLK,),
    in_specs=[pl.BlockSpec((BLK, F), lambda i: (i, 0)), _WFULL],
    out_specs=_ROW,
    out_shape=_HOUT,
)

_tc1b = pl.pallas_call(
    _tc1b_body,
    grid=(NP // BLK,),
    in_specs=[_ROW, _DEGP],
    out_specs=[_GHALF, _GHALF],
    out_shape=[_GOUT, _GOUT],
)

_tc2 = pl.pallas_call(
    _tc2_body,
    grid=(NP // BLK,),
    in_specs=[_ROW, _DEGP, _WFULL, _BROW],
    out_specs=[_GHALF, _GHALF],
    out_shape=[_GOUT, _GOUT],
)

_tc3 = pl.pallas_call(
    _tc3_body,
    grid=(NP // BLK,),
    in_specs=[_ROW, _DEGP, _BROW,
              pl.BlockSpec((F, 64), lambda i: (0, 0)),
              pl.BlockSpec((1, 64), lambda i: (0, 0))],
    out_specs=pl.BlockSpec((BLK, 64), lambda i: (i, 0)),
    out_shape=jax.ShapeDtypeStruct((NP, 64), jnp.float32),
)


def kernel(x, edge_index, hyperedge_index, W1, b1, Wh1, bh1, W2, b2, Wh2,
           bh2, lp_W, lp_b):
    del hyperedge_index

    wcat1 = jnp.concatenate([W1, Wh1], axis=1)
    wbd2 = jnp.zeros((F, F), jnp.float32)
    wbd2 = wbd2.at[:64, :64].set(W2).at[64:, 64:].set(Wh2)
    bcat1 = jnp.concatenate([b1, bh1])[None, :]
    bcat2 = jnp.concatenate([b2, bh2])[None, :]

    src = edge_index[0].reshape(NCHUNK, 128)
    dst = edge_index[1].reshape(NCHUNK, 128)

    sc_degree, sc_message_pass = _sc_kernels()
    degp = sc_degree(dst)

    h1 = _tc1a(x, wcat1)
    g1a, g1b = _tc1b(h1, degp)
    acc1 = sc_message_pass(g1a, g1b, src, dst)
    g2a, g2b = _tc2(acc1, degp, wbd2, bcat1)
    acc2 = sc_message_pass(g2a, g2b, src, dst)
    out = _tc3(acc2, degp, bcat2, lp_W, lp_b[None, :])
    return out[:N]


# speedup vs baseline: 47.1915x; 1.0400x over previous
import functools

import jax
import jax.numpy as jnp
from jax import lax
from jax.experimental import pallas as pl
from jax.experimental.pallas import tpu as pltpu
from jax.experimental.pallas import tpu_sc as plsc

N = 10000
E = 320000
F = 128
FH = 64
NP = 10240
NC = 2
NS = 16
NCHUNK = E // 128
CPT = NCHUNK // NS
CREM = NCHUNK - CPT * NS
CPW = NCHUNK // (NS * NC)
WREM = NCHUNK - CPW * NS * NC
RPT = NP // NS
BLK = 1024


def _deg_body(ei_hbm, degp, dst_v, ones_v, zrow_v, deg_sh):
    cid = lax.axis_index("c")
    sid = lax.axis_index("s")
    wid = sid * NC + cid
    for k in range(8):
        ones_v[pl.ds(k * 16, 16)] = jnp.full((16,), 1.0, jnp.float32)
    for k in range(RPT // 16):
        zrow_v[pl.ds(k * 16, 16)] = jnp.zeros((16,), jnp.float32)
    pltpu.sync_copy(ei_hbm.at[1, pl.ds(wid * CPW, CPW)],
                    dst_v.at[pl.ds(0, CPW)])

    @pl.when(wid < WREM)
    def _():
        pltpu.sync_copy(ei_hbm.at[1, NS * NC * CPW + wid], dst_v.at[CPW])

    pltpu.sync_copy(zrow_v, deg_sh.at[pl.ds(sid * RPT, RPT)])
    plsc.subcore_barrier()

    @pl.loop(0, CPW, step=2)
    def _(jo):
        for b in range(2):
            pltpu.sync_copy(ones_v, deg_sh.at[dst_v.at[jo + b]], add=True)

    @pl.when(wid < WREM)
    def _():
        pltpu.sync_copy(ones_v, deg_sh.at[dst_v.at[CPW]], add=True)

    plsc.subcore_barrier()
    pltpu.sync_copy(deg_sh.at[pl.ds(sid * RPT, RPT)],
                    degp.at[cid, pl.ds(sid * RPT, RPT)])


def _mp_pipeline(gsub, src_v, dst_v, ring, acc_sh, sid):
    pltpu.sync_copy(gsub.at[pl.ds(sid * RPT, RPT)],
                    acc_sh.at[pl.ds(sid * RPT, RPT)])
    plsc.subcore_barrier()

    for b in range(4):
        pltpu.async_copy(gsub.at[src_v.at[b]], ring[b][0], ring[b][1])

    @pl.loop(0, CPT, step=4)
    def _(jo):
        for b, (rows, sem) in enumerate(ring):
            j = jo + b
            pltpu.make_async_copy(gsub.at[src_v.at[j]], rows, sem).wait()
            pltpu.sync_copy(rows, acc_sh.at[dst_v.at[j]], add=True)

            @pl.when(j + 4 < CPT)
            def _():
                pltpu.async_copy(gsub.at[src_v.at[j + 4]], rows, sem)

    @pl.when(sid < CREM)
    def _():
        rows, sem = ring[0]
        pltpu.async_copy(gsub.at[src_v.at[CPT]], rows, sem).wait()
        pltpu.sync_copy(rows, acc_sh.at[dst_v.at[CPT]], add=True)


def _mp_body(g0_hbm, g1_hbm, ei_hbm, acc, src_v, dst_v, rows0,
             rows1, rows2, rows3, sem0, sem1, sem2, sem3, acc_sh):
    cid = lax.axis_index("c")
    sid = lax.axis_index("s")
    pltpu.sync_copy(ei_hbm.at[0, pl.ds(sid * CPT, CPT)],
                    src_v.at[pl.ds(0, CPT)])
    pltpu.sync_copy(ei_hbm.at[1, pl.ds(sid * CPT, CPT)],
                    dst_v.at[pl.ds(0, CPT)])

    @pl.when(sid < CREM)
    def _():
        pltpu.sync_copy(ei_hbm.at[0, NS * CPT + sid], src_v.at[CPT])
        pltpu.sync_copy(ei_hbm.at[1, NS * CPT + sid], dst_v.at[CPT])

    ring = ((rows0, sem0), (rows1, sem1), (rows2, sem2), (rows3, sem3))

    @pl.when(cid == 0)
    def _():
        _mp_pipeline(g0_hbm, src_v, dst_v, ring, acc_sh, sid)

    @pl.when(cid == 1)
    def _():
        _mp_pipeline(g1_hbm, src_v, dst_v, ring, acc_sh, sid)

    plsc.subcore_barrier()
    pltpu.sync_copy(acc_sh.at[pl.ds(sid * RPT, RPT)],
                    acc.at[pl.ds(sid * RPT, RPT), pl.ds(cid * FH, FH)])


@functools.cache
def _sc_kernels():
    mesh = plsc.VectorSubcoreMesh(core_axis_name="c", subcore_axis_name="s")
    sc_degree = pl.kernel(
        _deg_body,
        out_type=jax.ShapeDtypeStruct((NC, NP), jnp.float32),
        mesh=mesh,
        compiler_params=pltpu.CompilerParams(use_tc_tiling_on_sc=False),
        scratch_types=[
            pltpu.VMEM((CPW + 1, 128), jnp.int32),
            pltpu.VMEM((128,), jnp.float32),
            pltpu.VMEM((RPT,), jnp.float32),
            pltpu.VMEM_SHARED((NP,), jnp.float32),
        ],
    )
    sc_message_pass = pl.kernel(
        _mp_body,
        out_type=jax.ShapeDtypeStruct((NP, F), jnp.float32),
        mesh=mesh,
        compiler_params=pltpu.CompilerParams(use_tc_tiling_on_sc=False),
        scratch_types=[
            pltpu.VMEM((CPT + 1, 128), jnp.int32),
            pltpu.VMEM((CPT + 1, 128), jnp.int32),
            pltpu.VMEM((128, FH), jnp.float32),
            pltpu.VMEM((128, FH), jnp.float32),
            pltpu.VMEM((128, FH), jnp.float32),
            pltpu.VMEM((128, FH), jnp.float32),
            pltpu.SemaphoreType.DMA,
            pltpu.SemaphoreType.DMA,
            pltpu.SemaphoreType.DMA,
            pltpu.SemaphoreType.DMA,
            pltpu.VMEM_SHARED((NP, FH), jnp.float32),
        ],
    )
    return sc_degree, sc_message_pass


def _tc1a_body(x_ref, w_ref, h_ref):
    h_ref[...] = jnp.dot(x_ref[...], w_ref[...],
                         preferred_element_type=jnp.float32)


def _tc1b_body(h_ref, degp_ref, g0_ref, g1_ref):
    dinv = lax.rsqrt(1.0 + degp_ref[0, :] + degp_ref[1, :])
    g = h_ref[...] * dinv[:, None]
    g0_ref[...] = g[:, :FH]
    g1_ref[...] = g[:, FH:]


def _tc2_body(acc_ref, degp_ref, wbd_ref, b1_ref, g0_ref, g1_ref):
    dinv = lax.rsqrt(1.0 + degp_ref[0, :] + degp_ref[1, :])
    x1 = jnp.maximum(acc_ref[...] * dinv[:, None] + b1_ref[...], 0.0)
    h2 = jnp.dot(x1, wbd_ref[...], preferred_element_type=jnp.float32)
    g = h2 * dinv[:, None]
    g0_ref[...] = g[:, :FH]
    g1_ref[...] = g[:, FH:]


def _tc3_body(acc_ref, degp_ref, b2_ref, lpw_ref, lpb_ref, out_ref):
    dinv = lax.rsqrt(1.0 + degp_ref[0, :] + degp_ref[1, :])
    x2 = acc_ref[...] * dinv[:, None] + b2_ref[...]
    logits = jnp.dot(x2, lpw_ref[...],
                     preferred_element_type=jnp.float32) + lpb_ref[...]
    m = jnp.max(logits, axis=1, keepdims=True)
    lse = jnp.log(jnp.sum(jnp.exp(logits - m), axis=1, keepdims=True)) + m
    out_ref[...] = logits - lse


_ROW = pl.BlockSpec((BLK, F), lambda i: (i, 0))
_ACC = pl.BlockSpec((NC, BLK, FH), lambda i: (0, i, 0))
_DEGP = pl.BlockSpec((NC, BLK), lambda i: (0, i))
_WFULL = pl.BlockSpec((F, F), lambda i: (0, 0))
_BROW = pl.BlockSpec((1, F), lambda i: (0, 0))
_GHALF = pl.BlockSpec((BLK, FH), lambda i: (i, 0))
_HOUT = jax.ShapeDtypeStruct((NP, F), jnp.float32)
_GOUT = jax.ShapeDtypeStruct((NP, FH), jnp.float32)

_tc1a = pl.pallas_call(
    _tc1a_body,
    grid=(NP // BLK,),
    in_specs=[pl.BlockSpec((BLK, F), lambda i: (i, 0)), _WFULL],
    out_specs=_ROW,
    out_shape=_HOUT,
)

_tc1b = pl.pallas_call(
    _tc1b_body,
    grid=(NP // BLK,),
    in_specs=[_ROW, _DEGP],
    out_specs=[_GHALF, _GHALF],
    out_shape=[_GOUT, _GOUT],
)

_tc2 = pl.pallas_call(
    _tc2_body,
    grid=(NP // BLK,),
    in_specs=[_ROW, _DEGP, _WFULL, _BROW],
    out_specs=[_GHALF, _GHALF],
    out_shape=[_GOUT, _GOUT],
)

_tc3 = pl.pallas_call(
    _tc3_body,
    grid=(NP // BLK,),
    in_specs=[_ROW, _DEGP, _BROW,
              pl.BlockSpec((F, 64), lambda i: (0, 0)),
              pl.BlockSpec((1, 64), lambda i: (0, 0))],
    out_specs=pl.BlockSpec((BLK, 64), lambda i: (i, 0)),
    out_shape=jax.ShapeDtypeStruct((NP, 64), jnp.float32),
)


def kernel(x, edge_index, hyperedge_index, W1, b1, Wh1, bh1, W2, b2, Wh2,
           bh2, lp_W, lp_b):
    del hyperedge_index

    wcat1 = jnp.concatenate([W1, Wh1], axis=1)
    wbd2 = jnp.zeros((F, F), jnp.float32)
    wbd2 = wbd2.at[:64, :64].set(W2).at[64:, 64:].set(Wh2)
    bcat1 = jnp.concatenate([b1, bh1])[None, :]
    bcat2 = jnp.concatenate([b2, bh2])[None, :]

    ei3 = edge_index.reshape(2, NCHUNK, 128)

    sc_degree, sc_message_pass = _sc_kernels()
    degp = sc_degree(ei3)

    h1 = _tc1a(x, wcat1)
    g1a, g1b = _tc1b(h1, degp)
    acc1 = sc_message_pass(g1a, g1b, ei3)
    g2a, g2b = _tc2(acc1, degp, wbd2, bcat1)
    acc2 = sc_message_pass(g2a, g2b, ei3)
    out = _tc3(acc2, degp, bcat2, lp_W, lp_b[None, :])
    return out[:N]
